# baseline (speedup 1.0000x reference)
import functools
import jax
import jax.numpy as jnp
from jax import lax
from jax.experimental import pallas as pl
from jax.experimental.pallas import tpu as pltpu
from jax.experimental.pallas import tpu_sc as plsc

_N = 10000
_E = 320000
_D = 128
_NC = 2
_NS = 16
_NW = _NC * _NS
_EPW = _E // _NW
_SB = 80
_NB = _EPW // _SB
_GP = _SB // 16
_CHUNK = 80
_NCH = _N // _CHUNK

_mesh = plsc.VectorSubcoreMesh(core_axis_name="c", subcore_axis_name="s",
                               num_cores=_NC, num_subcores=_NS)


def _iota16():
    return lax.iota(jnp.int32, 16)


def _full16(v):
    return jnp.full((16,), v, jnp.int32)


def _sc_b1_body(src_hbm, dst_hbm, am_hbm, q_hbm, m_hbm,
                ex_hbm, s0_hbm, s1_hbm,
                stg_src, stg_dst, stg_am, qv, exv, zb, mv,
                dstb0, dstb1, s_sh, sem0, sem1):
    cid = lax.axis_index("c")
    sid = lax.axis_index("s")
    wid = sid * _NC + cid
    base = wid * _EPW
    dstbs = (dstb0, dstb1)
    sems = (sem0, sem1)

    pltpu.sync_copy(m_hbm, mv)
    pltpu.sync_copy(src_hbm.at[pl.ds(base, _EPW)], stg_src)
    pltpu.sync_copy(dst_hbm.at[pl.ds(base, _EPW)], stg_dst)
    pltpu.sync_copy(am_hbm.at[pl.ds(base, _EPW)], stg_am)
    pltpu.sync_copy(q_hbm, qv)

    def _z(i, _):
        zb[pl.ds(i * 16, 16)] = jnp.zeros((16,), jnp.float32)
        return 0
    lax.fori_loop(0, _CHUNK // 16, _z, 0)
    for j in range(_NCH):
        @pl.when(sid == (j % _NS))
        def _():
            pltpu.sync_copy(zb, s_sh.at[pl.ds(j * _CHUNK, _CHUNK)])
    plsc.subcore_barrier()

    def _grp(g, _):
        sl = pl.ds(g * 16, 16)
        qs = plsc.load_gather(qv, [stg_src[sl]])
        exv[sl] = jnp.exp(stg_am[sl] * qs - mv[...])
        return 0
    lax.fori_loop(0, _EPW // 16, _grp, 0)
    pltpu.sync_copy(exv, ex_hbm.at[pl.ds(base, _EPW)])

    def _fill(buf, bb):
        def _f(g, _):
            buf[pl.ds(g * 16, 16)] = stg_dst[pl.ds(bb * _SB + g * 16, 16)]
            return 0
        lax.fori_loop(0, _GP, _f, 0)

    def _pair(i, _):
        for s in range(2):
            bb = i * 2 + s

            @pl.when(i > 0)
            def _():
                pltpu.make_async_copy(
                    exv.at[pl.ds(0, _SB)], s_sh.at[dstbs[s]], sems[s]).wait()
            _fill(dstbs[s], bb)
            pltpu.async_copy(exv.at[pl.ds(bb * _SB, _SB)],
                             s_sh.at[dstbs[s]], sems[s], add=True)
        return 0
    lax.fori_loop(0, _NB // 2, _pair, 0)
    for s in range(2):
        pltpu.make_async_copy(
            exv.at[pl.ds(0, _SB)], s_sh.at[dstbs[s]], sems[s]).wait()
    _fill(dstb0, _NB - 1)
    pltpu.sync_copy(exv.at[pl.ds((_NB - 1) * _SB, _SB)],
                    s_sh.at[dstb0], add=True)

    plsc.subcore_barrier()
    for j in range(_NCH):
        @pl.when((sid == (j % _NS)) & (cid == 0))
        def _():
            pltpu.sync_copy(s_sh.at[pl.ds(j * _CHUNK, _CHUNK)], zb)
            pltpu.sync_copy(zb, s0_hbm.at[pl.ds(j * _CHUNK, _CHUNK)])

        @pl.when((sid == (j % _NS)) & (cid == 1))
        def _():
            pltpu.sync_copy(s_sh.at[pl.ds(j * _CHUNK, _CHUNK)], zb)
            pltpu.sync_copy(zb, s1_hbm.at[pl.ds(j * _CHUNK, _CHUNK)])


def _sc_b1(src, dst, amean, q, m16):
    return pl.kernel(
        _sc_b1_body,
        out_type=(
            jax.ShapeDtypeStruct((_E,), jnp.float32),
            jax.ShapeDtypeStruct((_N,), jnp.float32),
            jax.ShapeDtypeStruct((_N,), jnp.float32),
        ),
        mesh=_mesh,
        scratch_types=(
            pltpu.VMEM((_EPW,), jnp.int32),
            pltpu.VMEM((_EPW,), jnp.int32),
            pltpu.VMEM((_EPW,), jnp.float32),
            pltpu.VMEM((_N,), jnp.float32),
            pltpu.VMEM((_EPW,), jnp.float32),
            pltpu.VMEM((_CHUNK,), jnp.float32),
            pltpu.VMEM((16,), jnp.float32),
            pltpu.VMEM((_SB,), jnp.int32),
            pltpu.VMEM((_SB,), jnp.int32),
            pltpu.VMEM_SHARED((_N,), jnp.float32),
            pltpu.SemaphoreType.DMA,
            pltpu.SemaphoreType.DMA,
        ),
        compiler_params=pltpu.CompilerParams(needs_layout_passes=False),
    )(src, dst, amean, q, m16)


_NSLOT = 4
_FW = _SB * 32
_NR = _FW // 128


def _sc_b2_body(src_hbm, dst_hbm, ex_hbm,
                e0_hbm, e1_hbm, e2_hbm, e3_hbm,
                st_hbm, g_hbm,
                conv0_hbm, conv1_hbm,
                stot, rdf,
                msgf0, msgf1, idx0, idx1,
                grows0, grows1, grows2, grows3,
                srcb0, srcb1, srcb2, srcb3,
                dstb0, dstb1, dstb2, dstb3,
                exb0, exb1, exb2, exb3,
                ea00, ea01, ea02, ea03,
                ea10, ea11, ea12, ea13,
                ea20, ea21, ea22, ea23,
                ea30, ea31, ea32, ea33,
                conv_sh,
                gsem0, gsem1, gsem2, gsem3,
                isem0, isem1, isem2, isem3,
                fsem0, fsem1, fsem2, fsem3, wsem0, wsem1):
    cid = lax.axis_index("c")
    sid = lax.axis_index("s")
    wid = sid * _NC + cid
    base = wid * _EPW
    iota = _iota16()
    msgfs = (msgf0, msgf1)
    idxs = (idx0, idx1)
    wsems = (wsem0, wsem1)
    growss = (grows0, grows1, grows2, grows3)
    srcbs = (srcb0, srcb1, srcb2, srcb3)
    dstbs = (dstb0, dstb1, dstb2, dstb3)
    exbs = (exb0, exb1, exb2, exb3)
    eabs = ((ea00, ea10, ea20, ea30), (ea01, ea11, ea21, ea31),
            (ea02, ea12, ea22, ea32), (ea03, ea13, ea23, ea33))
    gsems = (gsem0, gsem1, gsem2, gsem3)
    isems = (isem0, isem1, isem2, isem3)
    fsems = (fsem0, fsem1, fsem2, fsem3)

    pltpu.sync_copy(st_hbm, stot)

    zero16 = jnp.zeros((16,), jnp.float32)

    def _zm(i, _):
        sl = pl.ds(i * 16, 16)
        msgf0[sl] = zero16
        msgf1[sl] = zero16
        rdf[sl] = zero16
        return 0
    lax.fori_loop(0, _FW // 16, _zm, 0)
    for j in range(_NB):
        @pl.when(sid == (j % _NS))
        def _():
            pltpu.sync_copy(rdf, conv_sh.at[pl.ds(j * _FW, _FW)])
    plsc.subcore_barrier()

    def _src_fetch(s, bb):
        pltpu.async_copy(src_hbm.at[pl.ds(base + bb * _SB, _SB)],
                         srcbs[s], isems[s])

    def _src_wait(s):
        pltpu.make_async_copy(src_hbm.at[pl.ds(base, _SB)],
                              srcbs[s], isems[s]).wait()

    def _in_fetch(s, bb):
        off = base + bb * _SB
        pltpu.async_copy(dst_hbm.at[pl.ds(off, _SB)], dstbs[s], fsems[s])
        pltpu.async_copy(ex_hbm.at[pl.ds(off, _SB)], exbs[s], fsems[s])
        for k, eh in enumerate((e0_hbm, e1_hbm, e2_hbm, e3_hbm)):
            pltpu.async_copy(eh.at[pl.ds(off, _SB)], eabs[s][k], fsems[s])

    def _in_wait(s):
        pltpu.make_async_copy(dst_hbm.at[pl.ds(base, _SB)],
                              dstbs[s], fsems[s]).wait()
        pltpu.make_async_copy(ex_hbm.at[pl.ds(base, _SB)],
                              exbs[s], fsems[s]).wait()
        for k, eh in enumerate((e0_hbm, e1_hbm, e2_hbm, e3_hbm)):
            pltpu.make_async_copy(eh.at[pl.ds(base, _SB)],
                                  eabs[s][k], fsems[s]).wait()

    def _scatter(s2):
        for r in range(_NR):
            pltpu.async_copy(msgfs[s2].at[pl.ds(r * 128, 128)],
                             conv_sh.at[idxs[s2].at[r]], wsems[s2], add=True)

    def _scatter_wait(s2):
        pltpu.make_async_copy(ex_hbm.at[pl.ds(0, _FW)], msgfs[s2],
                              wsems[s2]).wait()

    def _compute(s, s2):
        e0b, e1b, e2b, e3b = eabs[s]

        def _grp(g, _):
            sl = pl.ds(g * 16, 16)
            ev = g * 16 + iota
            pbase = g * (16 * 32) + iota * 32
            dv = dstbs[s][sl]
            dv32 = dv * 32
            stv = plsc.load_gather(stot, [dv])
            att = exbs[s][sl] / (stv + 1e-16)
            cf = att + 1.0
            w0 = e0b[sl] * cf
            w1 = e1b[sl] * cf
            w2 = e2b[sl] * cf
            w3 = e3b[sl] * cf
            for c in range(32):
                pv = pbase + c
                plsc.store_scatter(idxs[s2], [pv >> 7, pv & 127], dv32 + c)
                if c < 30:
                    acc = w0 * plsc.load_gather(growss[s], [ev, _full16(c)])
                    acc = acc + w1 * plsc.load_gather(growss[s],
                                                      [ev, _full16(32 + c)])
                    acc = acc + w2 * plsc.load_gather(growss[s],
                                                      [ev, _full16(64 + c)])
                    acc = acc + w3 * plsc.load_gather(growss[s],
                                                      [ev, _full16(96 + c)])
                    plsc.store_scatter(msgfs[s2], [pv], acc)
            return 0
        lax.fori_loop(0, _GP, _grp, 0)

    for s in range(_NSLOT):
        _src_fetch(s, s)
        _in_fetch(s, s)
    for s in range(_NSLOT):
        _src_wait(s)
        pltpu.async_copy(g_hbm.at[srcbs[s]], growss[s], gsems[s])

    def _round(i, _):
        for s in range(_NSLOT):
            bb = i * _NSLOT + s
            s2 = s % 2
            pltpu.make_async_copy(
                g_hbm.at[srcbs[s]], growss[s], gsems[s]).wait()

            @pl.when(bb + _NSLOT < _NB)
            def _():
                _src_fetch(s, bb + _NSLOT)
            _in_wait(s)

            @pl.when(bb > 1)
            def _():
                _scatter_wait(s2)

            @pl.when((bb > 0) & (bb + _NSLOT - 1 < _NB))
            def _():
                _in_fetch((s + _NSLOT - 1) % _NSLOT, bb + _NSLOT - 1)
            _compute(s, s2)
            _scatter(s2)

            @pl.when(bb + _NSLOT < _NB)
            def _():
                _src_wait(s)
                pltpu.async_copy(g_hbm.at[srcbs[s]], growss[s], gsems[s])
        return 0
    lax.fori_loop(0, _NB // _NSLOT, _round, 0)

    pltpu.make_async_copy(g_hbm.at[srcbs[0]], growss[0], gsems[0]).wait()
    _in_wait(0)
    _scatter_wait(0)
    _compute(0, 0)
    _scatter(0)
    _scatter_wait(0)
    _scatter_wait(1)

    plsc.subcore_barrier()
    for j in range(_NB):
        @pl.when((sid == (j % _NS)) & (cid == 0))
        def _():
            pltpu.sync_copy(conv_sh.at[pl.ds(j * _FW, _FW)], rdf)
            pltpu.sync_copy(rdf, conv0_hbm.at[pl.ds(j * _FW, _FW)])

        @pl.when((sid == (j % _NS)) & (cid == 1))
        def _():
            pltpu.sync_copy(conv_sh.at[pl.ds(j * _FW, _FW)], rdf)
            pltpu.sync_copy(rdf, conv1_hbm.at[pl.ds(j * _FW, _FW)])


def _sc_b2(src, dst, ex, ea0, ea1, ea2, ea3, s_tot, g_tab):
    sb_i32 = pltpu.VMEM((_SB,), jnp.int32)
    sb_f32 = pltpu.VMEM((_SB,), jnp.float32)
    gr = pltpu.VMEM((_SB, _D), jnp.float32)
    fw = pltpu.VMEM((_FW,), jnp.float32)
    ix = pltpu.VMEM((_NR, 128), jnp.int32)
    dma = pltpu.SemaphoreType.DMA
    return pl.kernel(
        _sc_b2_body,
        out_type=(jax.ShapeDtypeStruct((_N * 32,), jnp.float32),
                  jax.ShapeDtypeStruct((_N * 32,), jnp.float32)),
        mesh=_mesh,
        scratch_types=(
            (pltpu.VMEM((_N,), jnp.float32), fw, fw, fw, ix, ix)
            + (gr,) * 4
            + (sb_i32,) * 8 + (sb_f32,) * 20
            + (pltpu.VMEM_SHARED((_N * 32,), jnp.float32),)
            + (dma,) * 14
        ),
        compiler_params=pltpu.CompilerParams(needs_layout_passes=False),
    )(src, dst, ex, ea0, ea1, ea2, ea3, s_tot, g_tab)


def _ssum_body(a_ref, b_ref, o_ref):
    o_ref[...] = a_ref[...] + b_ref[...]


def _ssum_tc(s0, s1):
    spec = pl.BlockSpec((1, _N), lambda: (0, 0))
    return pl.pallas_call(
        _ssum_body,
        in_specs=[spec, spec],
        out_specs=spec,
        out_shape=jax.ShapeDtypeStruct((1, _N), jnp.float32),
    )(s0.reshape(1, _N), s1.reshape(1, _N)).reshape(_N)


_BE = 2560


def _emlp_body(eft_ref, *refs):
    x = eft_ref[...]
    for l in range(2):
        w1, w2, w3, w4 = refs[4 * l:4 * l + 4]
        ea_ref, am_ref, amax_ref = refs[8 + 3 * l:8 + 3 * l + 3]
        lin = jnp.maximum(jnp.dot(w1[...], x), 0.0)
        gat = jnp.tanh(jnp.dot(w2[...], x)) * jnp.tanh(jnp.dot(w3[...], x))
        cat = jnp.concatenate([lin, gat], axis=0)
        ea = jnp.maximum(jnp.dot(w4[...], cat), 0.0)
        ea_ref[...] = ea
        am = jnp.mean(ea, axis=0, keepdims=True)
        am_ref[...] = am
        bmax = jnp.max(am)

        @pl.when(pl.program_id(0) == 0)
        def _():
            amax_ref[0, 0] = bmax

        @pl.when(pl.program_id(0) > 0)
        def _():
            amax_ref[0, 0] = jnp.maximum(amax_ref[0, 0], bmax)


def _emlp(eft, wts):
    wspec = lambda shp: pl.BlockSpec(shp, lambda i: (0, 0))
    n_blk = _E // _BE
    outs = jax.ShapeDtypeStruct
    return pl.pallas_call(
        _emlp_body,
        grid=(n_blk,),
        in_specs=[pl.BlockSpec((4, _BE), lambda i: (0, i))] +
                 [wspec(w.shape) for w in wts],
        out_specs=[pl.BlockSpec((4, _BE), lambda i: (0, i)),
                   pl.BlockSpec((1, _BE), lambda i: (0, i)),
                   pl.BlockSpec((1, 1), lambda i: (0, 0),
                                memory_space=pltpu.SMEM)] * 2,
        out_shape=[outs((4, _E), jnp.float32),
                   outs((1, _E), jnp.float32),
                   outs((1, 1), jnp.float32)] * 2,
    )(eft, *wts)


_BN = 2000


def _node_body(h_ref, wi_ref, wcf_ref, attv_ref, ws1_ref, bs1_ref,
               ws2_ref, bs2_ref, wo2_ref, bout_ref,
               g_ref, q_ref, s_ref, qmax_ref):
    h = h_ref[...]
    hp = jnp.dot(h, wi_ref[...])
    g_ref[...] = jnp.dot(hp, wcf_ref[...])
    qcol = jnp.dot(hp, attv_ref[...])
    q_ref[...] = qcol
    sk = (jnp.tanh(jnp.dot(hp, ws1_ref[...]) + bs1_ref[...]) *
          jnp.tanh(jnp.dot(hp, ws2_ref[...]) + bs2_ref[...]))
    s_ref[...] = h + jnp.dot(sk, wo2_ref[...]) + bout_ref[...]
    bmax = jnp.max(qcol)

    @pl.when(pl.program_id(0) == 0)
    def _():
        qmax_ref[0, 0] = bmax

    @pl.when(pl.program_id(0) > 0)
    def _():
        qmax_ref[0, 0] = jnp.maximum(qmax_ref[0, 0], bmax)


def _node_tc(h, wi, wcf, attv, ws1, bs1, ws2, bs2, wo2, bout):
    wspec = lambda shp: pl.BlockSpec(shp, lambda i: (0, 0))
    outs = jax.ShapeDtypeStruct
    wts = (wi, wcf, attv, ws1, bs1, ws2, bs2, wo2, bout)
    return pl.pallas_call(
        _node_body,
        grid=(_N // _BN,),
        in_specs=[pl.BlockSpec((_BN, _D), lambda i: (i, 0))] +
                 [wspec(w.shape) for w in wts],
        out_specs=[pl.BlockSpec((_BN, _D), lambda i: (i, 0)),
                   pl.BlockSpec((_BN, 1), lambda i: (i, 0)),
                   pl.BlockSpec((_BN, _D), lambda i: (i, 0)),
                   pl.BlockSpec((1, 1), lambda i: (0, 0),
                                memory_space=pltpu.SMEM)],
        out_shape=[outs((_N, _D), jnp.float32),
                   outs((_N, 1), jnp.float32),
                   outs((_N, _D), jnp.float32),
                   outs((1, 1), jnp.float32)],
    )(h, *wts)


def _final_body(c0_ref, c1_ref, s_ref, bconv_ref, wo1_ref, out_ref):
    conv = c0_ref[...] + c1_ref[...]
    oc = jnp.maximum(conv + bconv_ref[...], 0.0)
    out_ref[...] = s_ref[...] + jnp.dot(oc, wo1_ref[...])


def _final_tc(c0, c1, s, bconvp, wo1p):
    wspec = lambda shp: pl.BlockSpec(shp, lambda i: (0, 0))
    return pl.pallas_call(
        _final_body,
        grid=(_N // _BN,),
        in_specs=[pl.BlockSpec((_BN, 32), lambda i: (i, 0)),
                  pl.BlockSpec((_BN, 32), lambda i: (i, 0)),
                  pl.BlockSpec((_BN, _D), lambda i: (i, 0)),
                  wspec(bconvp.shape), wspec(wo1p.shape)],
        out_specs=pl.BlockSpec((_BN, _D), lambda i: (i, 0)),
        out_shape=jax.ShapeDtypeStruct((_N, _D), jnp.float32),
    )(c0, c1, s, bconvp, wo1p)


def kernel(h, edge_index, edge_feat, params):
    src = edge_index[0]
    dst = edge_index[1]
    eft = edge_feat.T

    emlp_wts = []
    for p in params:
        emlp_wts += [p["We1"].T, p["We2"].T, p["We3"].T, p["We4"].T]
    ea_t0, am0, amax0, ea_t1, am1, amax1 = _emlp(eft, emlp_wts)
    ea_by_layer = [
        ([ea_t0[k] for k in range(4)], am0.reshape(_E), amax0),
        ([ea_t1[k] for k in range(4)], am1.reshape(_E), amax1),
    ]

    out = h
    for p, (eas, amean, amax) in zip(params, ea_by_layer):
        wcf = jnp.pad(jnp.transpose(p["Wconv"], (1, 0, 2)),
                      ((0, 0), (0, 0), (0, 2))).reshape(_D, 128)
        g_tab, qcol, s_base, qmax = _node_tc(
            out, p["Wi"], wcf, p["attv"], p["Ws1"], p["bs1"].reshape(1, 2),
            p["Ws2"], p["bs2"].reshape(1, 2), p["Wout"][30:, :],
            p["bout"].reshape(1, _D))
        q = qcol.reshape(_N)
        m = amax.reshape(()) * jnp.maximum(qmax.reshape(()), 0.0)
        m16 = jnp.full((16,), m, jnp.float32)

        ex, s0, s1 = _sc_b1(src, dst, amean, q, m16)
        s_tot = _ssum_tc(s0, s1)
        conv0, conv1 = _sc_b2(src, dst, ex,
                              eas[0], eas[1], eas[2], eas[3], s_tot, g_tab)
        conv0 = conv0.reshape(-1)[:_N * 32].reshape(_N, 32)
        conv1 = conv1.reshape(-1)[:_N * 32].reshape(_N, 32)
        bconvp = jnp.pad(p["bconv"], (0, 2)).reshape(1, 32)
        wo1p = jnp.pad(p["Wout"][:30, :], ((0, 2), (0, 0)))
        out = _final_tc(conv0, conv1, s_base, bconvp, wo1p)
    return out


# speedup vs baseline: 1.0955x; 1.0955x over previous
import functools
import jax
import jax.numpy as jnp
from jax import lax
from jax.experimental import pallas as pl
from jax.experimental.pallas import tpu as pltpu
from jax.experimental.pallas import tpu_sc as plsc

_N = 10000
_E = 320000
_D = 128
_NC = 2
_NS = 16
_NW = _NC * _NS
_EPW = _E // _NW
_SB = 80
_NB = _EPW // _SB
_GP = _SB // 16
_CHUNK = 80
_NCH = _N // _CHUNK

_mesh = plsc.VectorSubcoreMesh(core_axis_name="c", subcore_axis_name="s",
                               num_cores=_NC, num_subcores=_NS)


def _iota16():
    return lax.iota(jnp.int32, 16)


def _full16(v):
    return jnp.full((16,), v, jnp.int32)


def _sc_b1_body(src_hbm, dst_hbm, am_hbm, q_hbm, m_hbm,
                ex_hbm, s0_hbm, s1_hbm,
                stg_src, stg_dst, stg_am, qv, exv, zb, mv,
                dstb0, dstb1, s_sh, sem0, sem1):
    cid = lax.axis_index("c")
    sid = lax.axis_index("s")
    wid = sid * _NC + cid
    base = wid * _EPW
    dstbs = (dstb0, dstb1)
    sems = (sem0, sem1)

    pltpu.sync_copy(m_hbm, mv)
    pltpu.sync_copy(src_hbm.at[pl.ds(base, _EPW)], stg_src)
    pltpu.sync_copy(dst_hbm.at[pl.ds(base, _EPW)], stg_dst)
    pltpu.sync_copy(am_hbm.at[pl.ds(base, _EPW)], stg_am)
    pltpu.sync_copy(q_hbm, qv)

    def _z(i, _):
        zb[pl.ds(i * 16, 16)] = jnp.zeros((16,), jnp.float32)
        return 0
    lax.fori_loop(0, _CHUNK // 16, _z, 0)
    for j in range(_NCH):
        @pl.when(sid == (j % _NS))
        def _():
            pltpu.sync_copy(zb, s_sh.at[pl.ds(j * _CHUNK, _CHUNK)])
    plsc.subcore_barrier()

    def _grp(g, _):
        sl = pl.ds(g * 16, 16)
        qs = plsc.load_gather(qv, [stg_src[sl]])
        exv[sl] = jnp.exp(stg_am[sl] * qs - mv[...])
        return 0
    lax.fori_loop(0, _EPW // 16, _grp, 0)
    pltpu.sync_copy(exv, ex_hbm.at[pl.ds(base, _EPW)])

    def _fill(buf, bb):
        def _f(g, _):
            buf[pl.ds(g * 16, 16)] = stg_dst[pl.ds(bb * _SB + g * 16, 16)]
            return 0
        lax.fori_loop(0, _GP, _f, 0)

    def _pair(i, _):
        for s in range(2):
            bb = i * 2 + s

            @pl.when(i > 0)
            def _():
                pltpu.make_async_copy(
                    exv.at[pl.ds(0, _SB)], s_sh.at[dstbs[s]], sems[s]).wait()
            _fill(dstbs[s], bb)
            pltpu.async_copy(exv.at[pl.ds(bb * _SB, _SB)],
                             s_sh.at[dstbs[s]], sems[s], add=True)
        return 0
    lax.fori_loop(0, _NB // 2, _pair, 0)
    for s in range(2):
        pltpu.make_async_copy(
            exv.at[pl.ds(0, _SB)], s_sh.at[dstbs[s]], sems[s]).wait()
    _fill(dstb0, _NB - 1)
    pltpu.sync_copy(exv.at[pl.ds((_NB - 1) * _SB, _SB)],
                    s_sh.at[dstb0], add=True)

    plsc.subcore_barrier()
    for j in range(_NCH):
        @pl.when((sid == (j % _NS)) & (cid == 0))
        def _():
            pltpu.sync_copy(s_sh.at[pl.ds(j * _CHUNK, _CHUNK)], zb)
            pltpu.sync_copy(zb, s0_hbm.at[pl.ds(j * _CHUNK, _CHUNK)])

        @pl.when((sid == (j % _NS)) & (cid == 1))
        def _():
            pltpu.sync_copy(s_sh.at[pl.ds(j * _CHUNK, _CHUNK)], zb)
            pltpu.sync_copy(zb, s1_hbm.at[pl.ds(j * _CHUNK, _CHUNK)])


def _sc_b1(src, dst, amean, q, m16):
    return pl.kernel(
        _sc_b1_body,
        out_type=(
            jax.ShapeDtypeStruct((_E,), jnp.float32),
            jax.ShapeDtypeStruct((_N,), jnp.float32),
            jax.ShapeDtypeStruct((_N,), jnp.float32),
        ),
        mesh=_mesh,
        scratch_types=(
            pltpu.VMEM((_EPW,), jnp.int32),
            pltpu.VMEM((_EPW,), jnp.int32),
            pltpu.VMEM((_EPW,), jnp.float32),
            pltpu.VMEM((_N,), jnp.float32),
            pltpu.VMEM((_EPW,), jnp.float32),
            pltpu.VMEM((_CHUNK,), jnp.float32),
            pltpu.VMEM((16,), jnp.float32),
            pltpu.VMEM((_SB,), jnp.int32),
            pltpu.VMEM((_SB,), jnp.int32),
            pltpu.VMEM_SHARED((_N,), jnp.float32),
            pltpu.SemaphoreType.DMA,
            pltpu.SemaphoreType.DMA,
        ),
        compiler_params=pltpu.CompilerParams(needs_layout_passes=False),
    )(src, dst, amean, q, m16)


def _sc_b2_body(src_hbm, dst_hbm, ex_hbm,
                e0_hbm, e1_hbm, e2_hbm, e3_hbm,
                st_hbm, g_hbm,
                conv0_hbm, conv1_hbm,
                stot, grows0, grows1, msg,
                srcb0, srcb1, dstb0, dstb1, exb0, exb1,
                ea00, ea01, ea10, ea11, ea20, ea21, ea30, ea31,
                conv_sh,
                gsem0, gsem1, isem0, isem1, fsem0, fsem1, wsem):
    cid = lax.axis_index("c")
    sid = lax.axis_index("s")
    wid = sid * _NC + cid
    base = wid * _EPW
    iota = _iota16()
    growss = (grows0, grows1)
    srcbs = (srcb0, srcb1)
    dstbs = (dstb0, dstb1)
    exbs = (exb0, exb1)
    eabs = ((ea00, ea10, ea20, ea30), (ea01, ea11, ea21, ea31))
    gsems = (gsem0, gsem1)
    isems = (isem0, isem1)
    fsems = (fsem0, fsem1)

    pltpu.sync_copy(st_hbm, stot)

    def _zm(i, _):
        for c in range(_D // 16):
            msg[i, pl.ds(c * 16, 16)] = jnp.zeros((16,), jnp.float32)
        return 0
    lax.fori_loop(0, _SB, _zm, 0)
    for j in range(_NCH):
        @pl.when(sid == (j % _NS))
        def _():
            pltpu.sync_copy(msg, conv_sh.at[pl.ds(j * _CHUNK, _CHUNK)])
    plsc.subcore_barrier()

    def _src_fetch(s, bb):
        pltpu.async_copy(src_hbm.at[pl.ds(base + bb * _SB, _SB)],
                         srcbs[s], isems[s])

    def _src_wait(s):
        pltpu.make_async_copy(src_hbm.at[pl.ds(base, _SB)],
                              srcbs[s], isems[s]).wait()

    def _in_fetch(s, bb):
        off = base + bb * _SB
        pltpu.async_copy(dst_hbm.at[pl.ds(off, _SB)], dstbs[s], fsems[s])
        pltpu.async_copy(ex_hbm.at[pl.ds(off, _SB)], exbs[s], fsems[s])
        for k, eh in enumerate((e0_hbm, e1_hbm, e2_hbm, e3_hbm)):
            pltpu.async_copy(eh.at[pl.ds(off, _SB)], eabs[s][k], fsems[s])

    def _in_wait(s):
        pltpu.make_async_copy(dst_hbm.at[pl.ds(base, _SB)],
                              dstbs[s], fsems[s]).wait()
        pltpu.make_async_copy(ex_hbm.at[pl.ds(base, _SB)],
                              exbs[s], fsems[s]).wait()
        for k, eh in enumerate((e0_hbm, e1_hbm, e2_hbm, e3_hbm)):
            pltpu.make_async_copy(eh.at[pl.ds(base, _SB)],
                                  eabs[s][k], fsems[s]).wait()

    def _scatter_wait():
        pltpu.make_async_copy(msg, conv_sh.at[dstb0], wsem).wait()

    def _compute(s):
        e0b, e1b, e2b, e3b = eabs[s]

        def _grp(g, _):
            sl = pl.ds(g * 16, 16)
            ev = g * 16 + iota
            dv = dstbs[s][sl]
            stv = plsc.load_gather(stot, [dv])
            att = exbs[s][sl] / (stv + 1e-16)
            cf = att + 1.0
            w0 = e0b[sl] * cf
            w1 = e1b[sl] * cf
            w2 = e2b[sl] * cf
            w3 = e3b[sl] * cf
            for c in range(30):
                acc = w0 * plsc.load_gather(growss[s], [ev, _full16(c)])
                acc = acc + w1 * plsc.load_gather(growss[s],
                                                  [ev, _full16(32 + c)])
                acc = acc + w2 * plsc.load_gather(growss[s],
                                                  [ev, _full16(64 + c)])
                acc = acc + w3 * plsc.load_gather(growss[s],
                                                  [ev, _full16(96 + c)])
                plsc.store_scatter(msg, [ev, _full16(c)], acc)
            return 0
        lax.fori_loop(0, _GP, _grp, 0)

    for s in range(2):
        _src_fetch(s, s)
        _in_fetch(s, s)
    for s in range(2):
        _src_wait(s)
        pltpu.async_copy(g_hbm.at[srcbs[s]], growss[s], gsems[s])

    def _pair(i, _):
        for s in range(2):
            bb = i * 2 + s
            pltpu.make_async_copy(
                g_hbm.at[srcbs[s]], growss[s], gsems[s]).wait()

            @pl.when(bb + 2 < _NB)
            def _():
                _src_fetch(s, bb + 2)
            _in_wait(s)

            @pl.when(bb > 0)
            def _():
                _scatter_wait()

            @pl.when((bb > 0) & (bb + 1 < _NB))
            def _():
                _in_fetch(1 - s, bb + 1)
            _compute(s)
            pltpu.async_copy(msg, conv_sh.at[dstbs[s]], wsem, add=True)

            @pl.when(bb + 2 < _NB)
            def _():
                _src_wait(s)
                pltpu.async_copy(g_hbm.at[srcbs[s]], growss[s], gsems[s])
        return 0
    lax.fori_loop(0, _NB // 2, _pair, 0)

    pltpu.make_async_copy(g_hbm.at[srcbs[0]], growss[0], gsems[0]).wait()
    _in_wait(0)
    _scatter_wait()
    _compute(0)
    pltpu.sync_copy(msg, conv_sh.at[dstbs[0]], add=True)

    plsc.subcore_barrier()
    for j in range(_NCH):
        @pl.when((sid == (j % _NS)) & (cid == 0))
        def _():
            pltpu.sync_copy(conv_sh.at[pl.ds(j * _CHUNK, _CHUNK)], msg)
            pltpu.sync_copy(msg, conv0_hbm.at[pl.ds(j * _CHUNK, _CHUNK)])

        @pl.when((sid == (j % _NS)) & (cid == 1))
        def _():
            pltpu.sync_copy(conv_sh.at[pl.ds(j * _CHUNK, _CHUNK)], msg)
            pltpu.sync_copy(msg, conv1_hbm.at[pl.ds(j * _CHUNK, _CHUNK)])


def _sc_b2(src, dst, ex, ea0, ea1, ea2, ea3, s_tot, g_tab):
    sb_i32 = pltpu.VMEM((_SB,), jnp.int32)
    sb_f32 = pltpu.VMEM((_SB,), jnp.float32)
    return pl.kernel(
        _sc_b2_body,
        out_type=(jax.ShapeDtypeStruct((_N, _D), jnp.float32),
                  jax.ShapeDtypeStruct((_N, _D), jnp.float32)),
        mesh=_mesh,
        scratch_types=(
            pltpu.VMEM((_N,), jnp.float32),
            pltpu.VMEM((_SB, _D), jnp.float32),
            pltpu.VMEM((_SB, _D), jnp.float32),
            pltpu.VMEM((_SB, _D), jnp.float32),
            sb_i32, sb_i32, sb_i32, sb_i32, sb_f32, sb_f32,
            sb_f32, sb_f32, sb_f32, sb_f32, sb_f32, sb_f32, sb_f32, sb_f32,
            pltpu.VMEM_SHARED((_N, _D), jnp.float32),
            pltpu.SemaphoreType.DMA,
            pltpu.SemaphoreType.DMA,
            pltpu.SemaphoreType.DMA,
            pltpu.SemaphoreType.DMA,
            pltpu.SemaphoreType.DMA,
            pltpu.SemaphoreType.DMA,
            pltpu.SemaphoreType.DMA,
        ),
        compiler_params=pltpu.CompilerParams(needs_layout_passes=False),
    )(src, dst, ex, ea0, ea1, ea2, ea3, s_tot, g_tab)


def _ssum_body(a_ref, b_ref, o_ref):
    o_ref[...] = a_ref[...] + b_ref[...]


def _ssum_tc(s0, s1):
    spec = pl.BlockSpec((1, _N), lambda: (0, 0))
    return pl.pallas_call(
        _ssum_body,
        in_specs=[spec, spec],
        out_specs=spec,
        out_shape=jax.ShapeDtypeStruct((1, _N), jnp.float32),
    )(s0.reshape(1, _N), s1.reshape(1, _N)).reshape(_N)


_BE = 2560


def _emlp_body(eft_ref, *refs):
    x = eft_ref[...]
    for l in range(2):
        w1, w2, w3, w4 = refs[4 * l:4 * l + 4]
        ea_ref, am_ref, amax_ref = refs[8 + 3 * l:8 + 3 * l + 3]
        lin = jnp.maximum(jnp.dot(w1[...], x), 0.0)
        gat = jnp.tanh(jnp.dot(w2[...], x)) * jnp.tanh(jnp.dot(w3[...], x))
        cat = jnp.concatenate([lin, gat], axis=0)
        ea = jnp.maximum(jnp.dot(w4[...], cat), 0.0)
        ea_ref[...] = ea
        am = jnp.mean(ea, axis=0, keepdims=True)
        am_ref[...] = am
        bmax = jnp.max(am)

        @pl.when(pl.program_id(0) == 0)
        def _():
            amax_ref[0, 0] = bmax

        @pl.when(pl.program_id(0) > 0)
        def _():
            amax_ref[0, 0] = jnp.maximum(amax_ref[0, 0], bmax)


def _emlp(eft, wts):
    wspec = lambda shp: pl.BlockSpec(shp, lambda i: (0, 0))
    n_blk = _E // _BE
    outs = jax.ShapeDtypeStruct
    return pl.pallas_call(
        _emlp_body,
        grid=(n_blk,),
        in_specs=[pl.BlockSpec((4, _BE), lambda i: (0, i))] +
                 [wspec(w.shape) for w in wts],
        out_specs=[pl.BlockSpec((4, _BE), lambda i: (0, i)),
                   pl.BlockSpec((1, _BE), lambda i: (0, i)),
                   pl.BlockSpec((1, 1), lambda i: (0, 0),
                                memory_space=pltpu.SMEM)] * 2,
        out_shape=[outs((4, _E), jnp.float32),
                   outs((1, _E), jnp.float32),
                   outs((1, 1), jnp.float32)] * 2,
    )(eft, *wts)


_BN = 2000


def _node_body(h_ref, wi_ref, wcf_ref, attv_ref, ws1_ref, bs1_ref,
               ws2_ref, bs2_ref, wo2_ref, bout_ref,
               g_ref, q_ref, s_ref, qmax_ref):
    h = h_ref[...]
    hp = jnp.dot(h, wi_ref[...])
    g_ref[...] = jnp.dot(hp, wcf_ref[...])
    qcol = jnp.dot(hp, attv_ref[...])
    q_ref[...] = qcol
    sk = (jnp.tanh(jnp.dot(hp, ws1_ref[...]) + bs1_ref[...]) *
          jnp.tanh(jnp.dot(hp, ws2_ref[...]) + bs2_ref[...]))
    s_ref[...] = h + jnp.dot(sk, wo2_ref[...]) + bout_ref[...]
    bmax = jnp.max(qcol)

    @pl.when(pl.program_id(0) == 0)
    def _():
        qmax_ref[0, 0] = bmax

    @pl.when(pl.program_id(0) > 0)
    def _():
        qmax_ref[0, 0] = jnp.maximum(qmax_ref[0, 0], bmax)


def _node_tc(h, wi, wcf, attv, ws1, bs1, ws2, bs2, wo2, bout):
    wspec = lambda shp: pl.BlockSpec(shp, lambda i: (0, 0))
    outs = jax.ShapeDtypeStruct
    wts = (wi, wcf, attv, ws1, bs1, ws2, bs2, wo2, bout)
    return pl.pallas_call(
        _node_body,
        grid=(_N // _BN,),
        in_specs=[pl.BlockSpec((_BN, _D), lambda i: (i, 0))] +
                 [wspec(w.shape) for w in wts],
        out_specs=[pl.BlockSpec((_BN, _D), lambda i: (i, 0)),
                   pl.BlockSpec((_BN, 1), lambda i: (i, 0)),
                   pl.BlockSpec((_BN, _D), lambda i: (i, 0)),
                   pl.BlockSpec((1, 1), lambda i: (0, 0),
                                memory_space=pltpu.SMEM)],
        out_shape=[outs((_N, _D), jnp.float32),
                   outs((_N, 1), jnp.float32),
                   outs((_N, _D), jnp.float32),
                   outs((1, 1), jnp.float32)],
    )(h, *wts)


def _final_body(c0_ref, c1_ref, s_ref, bconv_ref, wo1_ref, out_ref):
    conv = c0_ref[...] + c1_ref[...]
    oc = jnp.maximum(conv + bconv_ref[...], 0.0)
    out_ref[...] = s_ref[...] + jnp.dot(oc, wo1_ref[...])


def _final_tc(c0, c1, s, bconvp, wo1p):
    wspec = lambda shp: pl.BlockSpec(shp, lambda i: (0, 0))
    return pl.pallas_call(
        _final_body,
        grid=(_N // _BN,),
        in_specs=[pl.BlockSpec((_BN, _D), lambda i: (i, 0)),
                  pl.BlockSpec((_BN, _D), lambda i: (i, 0)),
                  pl.BlockSpec((_BN, _D), lambda i: (i, 0)),
                  wspec(bconvp.shape), wspec(wo1p.shape)],
        out_specs=pl.BlockSpec((_BN, _D), lambda i: (i, 0)),
        out_shape=jax.ShapeDtypeStruct((_N, _D), jnp.float32),
    )(c0, c1, s, bconvp, wo1p)


def kernel(h, edge_index, edge_feat, params):
    src = edge_index[0]
    dst = edge_index[1]
    eft = edge_feat.T

    emlp_wts = []
    for p in params:
        emlp_wts += [p["We1"].T, p["We2"].T, p["We3"].T, p["We4"].T]
    ea_t0, am0, amax0, ea_t1, am1, amax1 = _emlp(eft, emlp_wts)
    ea_by_layer = [
        ([ea_t0[k] for k in range(4)], am0.reshape(_E), amax0),
        ([ea_t1[k] for k in range(4)], am1.reshape(_E), amax1),
    ]

    out = h
    for p, (eas, amean, amax) in zip(params, ea_by_layer):
        wcf = jnp.pad(jnp.transpose(p["Wconv"], (1, 0, 2)),
                      ((0, 0), (0, 0), (0, 2))).reshape(_D, 128)
        g_tab, qcol, s_base, qmax = _node_tc(
            out, p["Wi"], wcf, p["attv"], p["Ws1"], p["bs1"].reshape(1, 2),
            p["Ws2"], p["bs2"].reshape(1, 2), p["Wout"][30:, :],
            p["bout"].reshape(1, _D))
        q = qcol.reshape(_N)
        m = amax.reshape(()) * jnp.maximum(qmax.reshape(()), 0.0)
        m16 = jnp.full((16,), m, jnp.float32)

        ex, s0, s1 = _sc_b1(src, dst, amean, q, m16)
        s_tot = _ssum_tc(s0, s1)
        conv0, conv1 = _sc_b2(src, dst, ex,
                              eas[0], eas[1], eas[2], eas[3], s_tot, g_tab)
        bconvp = jnp.pad(p["bconv"], (0, _D - 30)).reshape(1, _D)
        wo1p = jnp.pad(p["Wout"][:30, :], ((0, _D - 30), (0, 0)))
        out = _final_tc(conv0, conv1, s_base, bconvp, wo1p)
    return out
